# baseline (speedup 1.0000x reference)
import functools

import jax
import jax.numpy as jnp
from jax import lax
from jax.experimental import pallas as pl
from jax.experimental.pallas import tpu as pltpu
from jax.experimental.pallas import tpu_sc as plsc

BATCH = 4096
FIELDS = 26
EMBED_DIM = 16
NUM_CORES = 2
NUM_SUBCORES = 16
NUM_TILES = NUM_CORES * NUM_SUBCORES
B_PER_TILE = BATCH // NUM_TILES
VALS_PER_TILE = B_PER_TILE * FIELDS
GATHER_CHUNK = 128
NUM_CHUNKS = VALS_PER_TILE // GATHER_CHUNK
WORDS_PER_TILE = VALS_PER_TILE * EMBED_DIM


def _fm_body(sparse_hbm, embed_hbm, w_hbm, out_hbm,
             idx_v, gath_v, emb_v, so_v, out_v, sem_e, sem_g):
    wid = lax.axis_index("s") * NUM_CORES + lax.axis_index("c")
    b0 = wid * B_PER_TILE

    cp_e = pltpu.async_copy(
        embed_hbm.at[pl.ds(wid * WORDS_PER_TILE, WORDS_PER_TILE)], emb_v, sem_e)
    pltpu.sync_copy(sparse_hbm.at[wid], idx_v)
    gather_cps = []
    for r in range(NUM_CHUNKS):
        gather_cps.append(
            pltpu.async_copy(
                w_hbm.at[idx_v.at[r]],
                gath_v.at[pl.ds(r * GATHER_CHUNK, GATHER_CHUNK)],
                sem_g,
            )
        )

    cp_e.wait()

    def b_body(b, _):
        base = b * (FIELDS * EMBED_DIM)
        s = emb_v[pl.ds(base, EMBED_DIM)]
        ss = s * s
        for f in range(1, FIELDS):
            v = emb_v[pl.ds(base + f * EMBED_DIM, EMBED_DIM)]
            s = s + v
            ss = ss + v * v
        so_v[pl.ds(b * EMBED_DIM, EMBED_DIM)] = 0.5 * (s * s - ss)
        return 0

    lax.fori_loop(0, B_PER_TILE, b_body, 0)

    for cp in gather_cps:
        cp.wait()

    def c_body(c, _):
        lanes = lax.iota(jnp.int32, 16)
        pos_w = c * (16 * FIELDS) + lanes * FIELDS
        acc = plsc.load_gather(gath_v, [pos_w])
        for f in range(1, FIELDS):
            acc = acc + plsc.load_gather(gath_v, [pos_w + f])
        pos_s = c * (16 * EMBED_DIM) + lanes * EMBED_DIM
        for d in range(EMBED_DIM):
            acc = acc + plsc.load_gather(so_v, [pos_s + d])
        out_v[pl.ds(c * 16, 16)] = acc
        return 0

    lax.fori_loop(0, B_PER_TILE // 16, c_body, 0)

    pltpu.sync_copy(out_v, out_hbm.at[pl.ds(b0, B_PER_TILE)])


@jax.jit
def _fm_kernel(sparse_r, embed_inputs, w_flat):
    run = pl.kernel(
        _fm_body,
        out_type=jax.ShapeDtypeStruct((BATCH,), jnp.float32),
        mesh=plsc.VectorSubcoreMesh(core_axis_name="c", subcore_axis_name="s",
                                    num_cores=NUM_CORES,
                                    num_subcores=NUM_SUBCORES),
        scratch_types=[
            pltpu.VMEM((NUM_CHUNKS, GATHER_CHUNK), jnp.int32),
            pltpu.VMEM((VALS_PER_TILE,), jnp.float32),
            pltpu.VMEM((WORDS_PER_TILE,), jnp.float32),
            pltpu.VMEM((B_PER_TILE * EMBED_DIM,), jnp.float32),
            pltpu.VMEM((B_PER_TILE,), jnp.float32),
            pltpu.SemaphoreType.DMA,
            pltpu.SemaphoreType.DMA,
        ],
        compiler_params=pltpu.CompilerParams(needs_layout_passes=False),
    )
    return run(sparse_r, embed_inputs, w_flat)


def kernel(sparse_inputs, embed_inputs, w):
    sparse_r = sparse_inputs.reshape(NUM_TILES, NUM_CHUNKS, GATHER_CHUNK)
    embed_r = embed_inputs.reshape(-1)
    w_flat = w.reshape(-1)
    out = _fm_kernel(sparse_r, embed_r, w_flat)
    return out.reshape(BATCH, 1)


# speedup vs baseline: 1.5533x; 1.5533x over previous
import jax
import jax.numpy as jnp
from jax import lax
from jax.experimental import pallas as pl
from jax.experimental.pallas import tpu as pltpu
from jax.experimental.pallas import tpu_sc as plsc

BATCH = 4096
FIELDS = 26
EMBED_DIM = 16
NUM_CORES = 2
NUM_SUBCORES = 16
NUM_TILES = NUM_CORES * NUM_SUBCORES
B_PER_TILE = BATCH // NUM_TILES
EMB_ROWS = FIELDS * EMBED_DIM


def _fm_body(sparse_hbm, embed_hbm, w_hbm, out_hbm,
             idx_v, gath_v, emb_v, out_v, sem_e, sem_g):
    wid = lax.axis_index("s") * NUM_CORES + lax.axis_index("c")
    b0 = wid * B_PER_TILE

    cp_e = pltpu.async_copy(embed_hbm.at[:, pl.ds(b0, B_PER_TILE)],
                            emb_v, sem_e)
    pltpu.sync_copy(sparse_hbm.at[:, pl.ds(b0, B_PER_TILE)], idx_v)
    gather_cps = []
    for f in range(FIELDS):
        gather_cps.append(
            pltpu.async_copy(w_hbm.at[idx_v.at[f]], gath_v.at[f], sem_g))

    cp_e.wait()
    for cp in gather_cps:
        cp.wait()

    def c_body(c, _):
        col = c * 16
        ssq = emb_v[0, pl.ds(col, 16)] * 0.0
        sos = ssq
        for d in range(EMBED_DIM):
            v = emb_v[d, pl.ds(col, 16)]
            s = v
            ssq = ssq + v * v
            for f in range(1, FIELDS):
                v = emb_v[f * EMBED_DIM + d, pl.ds(col, 16)]
                s = s + v
                ssq = ssq + v * v
            sos = sos + s * s
        first = gath_v[0, pl.ds(col, 16)]
        for f in range(1, FIELDS):
            first = first + gath_v[f, pl.ds(col, 16)]
        out_v[0, pl.ds(col, 16)] = first + 0.5 * (sos - ssq)
        return 0

    lax.fori_loop(0, B_PER_TILE // 16, c_body, 0)

    pltpu.sync_copy(out_v, out_hbm.at[wid])


@jax.jit
def _fm_kernel(sparse_t, embed_t, w_flat):
    run = pl.kernel(
        _fm_body,
        out_type=jax.ShapeDtypeStruct((NUM_TILES, 1, B_PER_TILE), jnp.float32),
        mesh=plsc.VectorSubcoreMesh(core_axis_name="c", subcore_axis_name="s",
                                    num_cores=NUM_CORES,
                                    num_subcores=NUM_SUBCORES),
        scratch_types=[
            pltpu.VMEM((FIELDS, B_PER_TILE), jnp.int32),
            pltpu.VMEM((FIELDS, B_PER_TILE), jnp.float32),
            pltpu.VMEM((EMB_ROWS, B_PER_TILE), jnp.float32),
            pltpu.VMEM((1, B_PER_TILE), jnp.float32),
            pltpu.SemaphoreType.DMA,
            pltpu.SemaphoreType.DMA,
        ],
        compiler_params=pltpu.CompilerParams(needs_layout_passes=False),
    )
    return run(sparse_t, embed_t, w_flat)


def kernel(sparse_inputs, embed_inputs, w):
    sparse_t = sparse_inputs.T
    embed_t = jnp.transpose(embed_inputs, (1, 2, 0)).reshape(EMB_ROWS, BATCH)
    w_flat = w.reshape(-1)
    out = _fm_kernel(sparse_t, embed_t, w_flat)
    return out.reshape(BATCH, 1)


# speedup vs baseline: 3.7134x; 2.3907x over previous
import jax
import jax.numpy as jnp
from jax import lax
from jax.experimental import pallas as pl
from jax.experimental.pallas import tpu as pltpu
from jax.experimental.pallas import tpu_sc as plsc

BATCH = 4096
FIELDS = 26
EMBED_DIM = 16
NUM_CORES = 2
NUM_SUBCORES = 16
NUM_TILES = NUM_CORES * NUM_SUBCORES
B_PER_TILE = BATCH // NUM_TILES
EMB_ROWS = FIELDS * EMBED_DIM


def _fm_body(sparse_hbm, embed_hbm, w_hbm, out_hbm,
             idx_v, gath_v, emb_v, out_v, sem_e, sem_g):
    wid = lax.axis_index("s") * NUM_CORES + lax.axis_index("c")
    b0 = wid * B_PER_TILE

    cp_e = pltpu.async_copy(embed_hbm.at[:, pl.ds(b0, B_PER_TILE)],
                            emb_v, sem_e)
    pltpu.sync_copy(sparse_hbm.at[:, pl.ds(b0, B_PER_TILE)], idx_v)
    gather_cps = []
    for f in range(FIELDS):
        gather_cps.append(
            pltpu.async_copy(w_hbm.at[0].at[idx_v.at[f]], gath_v.at[f],
                             sem_g))

    cp_e.wait()

    def c_body(c, _):
        col = c * 16
        ssq = emb_v[0, pl.ds(col, 16)] * 0.0
        sos = ssq
        for d in range(EMBED_DIM):
            v = emb_v[d, pl.ds(col, 16)]
            s = v
            ssq = ssq + v * v
            for f in range(1, FIELDS):
                v = emb_v[f * EMBED_DIM + d, pl.ds(col, 16)]
                s = s + v
                ssq = ssq + v * v
            sos = sos + s * s
        out_v[0, pl.ds(col, 16)] = 0.5 * (sos - ssq)
        return 0

    lax.fori_loop(0, B_PER_TILE // 16, c_body, 0)

    for cp in gather_cps:
        cp.wait()

    def a_body(c, _):
        col = c * 16
        first = gath_v[0, pl.ds(col, 16)]
        for f in range(1, FIELDS):
            first = first + gath_v[f, pl.ds(col, 16)]
        out_v[0, pl.ds(col, 16)] = out_v[0, pl.ds(col, 16)] + first
        return 0

    lax.fori_loop(0, B_PER_TILE // 16, a_body, 0)

    pltpu.sync_copy(out_v, out_hbm.at[wid])


@jax.jit
def _fm_kernel(sparse_t, embed_t, w_flat):
    run = pl.kernel(
        _fm_body,
        out_type=jax.ShapeDtypeStruct((NUM_TILES, 1, B_PER_TILE), jnp.float32),
        mesh=plsc.VectorSubcoreMesh(core_axis_name="c", subcore_axis_name="s",
                                    num_cores=NUM_CORES,
                                    num_subcores=NUM_SUBCORES),
        scratch_types=[
            pltpu.VMEM((FIELDS, B_PER_TILE), jnp.int32),
            pltpu.VMEM((FIELDS, B_PER_TILE), jnp.float32),
            pltpu.VMEM((EMB_ROWS, B_PER_TILE), jnp.float32),
            pltpu.VMEM((1, B_PER_TILE), jnp.float32),
            pltpu.SemaphoreType.DMA,
            pltpu.SemaphoreType.DMA,
        ],
        compiler_params=pltpu.CompilerParams(needs_layout_passes=False),
    )
    return run(sparse_t, embed_t, w_flat)


def kernel(sparse_inputs, embed_inputs, w):
    sparse_t = sparse_inputs.T
    embed_t = jnp.transpose(embed_inputs, (1, 2, 0)).reshape(EMB_ROWS, BATCH)
    w_t = w.T
    out = _fm_kernel(sparse_t, embed_t, w_t)
    return out.reshape(BATCH, 1)
